# baseline (speedup 1.0000x reference)
import functools
import math

import jax
import jax.numpy as jnp
from jax import lax
from jax.experimental import pallas as pl
from jax.experimental.pallas import tpu as pltpu

_M = 0.5
_COS_M = math.cos(_M)
_SIN_M = math.sin(_M)
_TH = math.cos(math.pi - _M)
_MM = math.sin(math.pi - _M) * _M
_EPS = 1e-07

_GR = 8
_LW = 128
_CH = 8
_NBUF = 4


def _phi(ct):
    ct = jnp.clip(ct, -1.0 + _EPS, 1.0 - _EPS)
    sine = jnp.sqrt(1.0 - ct * ct)
    phi = ct * _COS_M - sine * _SIN_M
    return jnp.where(ct > _TH, phi, ct - _MM)


def _gather_body(lab_sref, lab_ref, *refs):
    xs, phi_ref = refs[:-1], refs[-1]
    rows = [xs[k][pl.ds(k, 1), :] for k in range(_GR)]
    m = jnp.concatenate(rows, axis=0)
    lab = lab_ref[...]
    lane = lax.broadcasted_iota(jnp.int32, m.shape, 1)
    mask = lane == lab % _LW
    ct = jnp.sum(jnp.where(mask, m, 0.0), axis=1, keepdims=True)
    phi_ref[...] = _phi(ct)


def _win_map(k, i, lab_sref):
    return (i, lab_sref[i * _GR + k] // _LW)


def _stream_body(s_ref, lab_ref, phi_ref, x_hbm, o_hbm, bin_, bout, sin, sout):
    n_rows = x_hbm.shape[0]
    n_chunks = n_rows // _CH

    n_cols = x_hbm.shape[1]

    def in_cp(c, k):
        return pltpu.make_async_copy(
            x_hbm.at[pl.ds(c * _CH, _CH), :], bin_.at[k], sin.at[k])

    def out_cp(c, k):
        return pltpu.make_async_copy(
            bout.at[k], o_hbm.at[pl.ds(c * _CH, _CH), :], sout.at[k])

    for k in range(_NBUF):
        in_cp(k, k).start()

    s = s_ref[0, 0]
    lane = lax.broadcasted_iota(jnp.int32, (1, _LW), 1)
    n_full = (n_cols // _LW) * _LW
    tail = n_cols - n_full
    lane_t = (
        lax.broadcasted_iota(jnp.int32, (1, tail), 1) if tail else None)

    def outer(o, carry):
        for k in range(_NBUF):
            c = o * _NBUF + k
            in_cp(c, k).wait()

            @pl.when(o > 0)
            def _():
                out_cp(c - _NBUF, k).wait()

            bout[k] = bin_[k] * s
            for r in range(_CH):
                lb = lab_ref[c * _CH + r]
                ph = phi_ref[c * _CH + r] * s
                w0 = jnp.minimum(lb // _LW, n_cols // _LW - 1) * _LW
                win = bout[k, pl.ds(r, 1), pl.ds(w0, _LW)]
                fix = jnp.where(lane == lb - w0, ph, win)
                bout[k, pl.ds(r, 1), pl.ds(w0, _LW)] = fix
                if tail:
                    @pl.when(lb >= n_full)
                    def _():
                        winq = bout[k, pl.ds(r, 1), pl.ds(n_full, tail)]
                        fixq = jnp.where(lane_t == lb - n_full, ph, winq)
                        bout[k, pl.ds(r, 1), pl.ds(n_full, tail)] = fixq
            out_cp(c, k).start()

            @pl.when(c + _NBUF < n_chunks)
            def _():
                in_cp(c + _NBUF, k).start()

        return carry

    lax.fori_loop(0, n_chunks // _NBUF, outer, 0)
    for k in range(_NBUF):
        out_cp(n_chunks - _NBUF + k, k).wait()


def kernel(cosine, labels, s):
    n_rows, n_cols = cosine.shape
    lab = labels.astype(jnp.int32)
    lab2d = lab.reshape(n_rows, 1)
    s_arr = jnp.asarray(s, jnp.float32).reshape(1, 1)

    phi = pl.pallas_call(
        _gather_body,
        grid_spec=pltpu.PrefetchScalarGridSpec(
            num_scalar_prefetch=1,
            grid=(n_rows // _GR,),
            in_specs=[pl.BlockSpec((_GR, 1), lambda i, ls: (i, 0))]
            + [
                pl.BlockSpec((_GR, _LW), functools.partial(_win_map, k))
                for k in range(_GR)
            ],
            out_specs=pl.BlockSpec((_GR, 1), lambda i, ls: (i, 0)),
        ),
        out_shape=jax.ShapeDtypeStruct((n_rows, 1), jnp.float32),
        compiler_params=pltpu.CompilerParams(
            dimension_semantics=("arbitrary",),
        ),
    )(lab, lab2d, *([cosine] * _GR))

    return pl.pallas_call(
        _stream_body,
        in_specs=[
            pl.BlockSpec(memory_space=pltpu.SMEM),
            pl.BlockSpec(memory_space=pltpu.SMEM),
            pl.BlockSpec(memory_space=pltpu.SMEM),
            pl.BlockSpec(memory_space=pltpu.HBM),
        ],
        out_specs=pl.BlockSpec(memory_space=pltpu.HBM),
        out_shape=jax.ShapeDtypeStruct((n_rows, n_cols), cosine.dtype),
        scratch_shapes=[
            pltpu.VMEM((_NBUF, _CH, n_cols), jnp.float32),
            pltpu.VMEM((_NBUF, _CH, n_cols), jnp.float32),
            pltpu.SemaphoreType.DMA((_NBUF,)),
            pltpu.SemaphoreType.DMA((_NBUF,)),
        ],
    )(s_arr, lab, phi.reshape(n_rows), cosine)


# speedup vs baseline: 1.0810x; 1.0810x over previous
import math

import jax
import jax.numpy as jnp
from jax import lax
from jax.experimental import pallas as pl
from jax.experimental.pallas import tpu as pltpu

_M = 0.5
_COS_M = math.cos(_M)
_SIN_M = math.sin(_M)
_TH = math.cos(math.pi - _M)
_MM = math.sin(math.pi - _M) * _M
_EPS = 1e-07

_LW = 128
_CH = 8
_NBUF = 4


def _phi(ct):
    ct = jnp.clip(ct, -1.0 + _EPS, 1.0 - _EPS)
    sine = jnp.sqrt(1.0 - ct * ct)
    phi = ct * _COS_M - sine * _SIN_M
    return jnp.where(ct > _TH, phi, ct - _MM)


def _stream_body(s_ref, lab_ref, x_hbm, o_hbm, bin_, bout, sin, sout):
    n_rows, n_cols = x_hbm.shape
    n_chunks = n_rows // _CH

    def in_cp(c, k):
        return pltpu.make_async_copy(
            x_hbm.at[pl.ds(c * _CH, _CH), :], bin_.at[k], sin.at[k])

    def out_cp(c, k):
        return pltpu.make_async_copy(
            bout.at[k], o_hbm.at[pl.ds(c * _CH, _CH), :], sout.at[k])

    for k in range(_NBUF):
        in_cp(k, k).start()

    s = s_ref[0, 0]
    lane = lax.broadcasted_iota(jnp.int32, (1, _LW), 1)
    n_full = (n_cols // _LW) * _LW
    tail = n_cols - n_full
    lane_t = (
        lax.broadcasted_iota(jnp.int32, (1, tail), 1) if tail else None)

    def outer(o, carry):
        for k in range(_NBUF):
            c = o * _NBUF + k
            in_cp(c, k).wait()

            @pl.when(o > 0)
            def _():
                out_cp(c - _NBUF, k).wait()

            bout[k] = bin_[k] * s
            for r in range(_CH):
                lb = lab_ref[c * _CH + r]
                w0 = jnp.minimum(lb // _LW, n_cols // _LW - 1) * _LW
                src = bin_[k, pl.ds(r, 1), pl.ds(w0, _LW)]
                win = bout[k, pl.ds(r, 1), pl.ds(w0, _LW)]
                fix = jnp.where(lane == lb - w0, _phi(src) * s, win)
                bout[k, pl.ds(r, 1), pl.ds(w0, _LW)] = fix
                if tail:
                    @pl.when(lb >= n_full)
                    def _():
                        srcq = bin_[k, pl.ds(r, 1), pl.ds(n_full, tail)]
                        winq = bout[k, pl.ds(r, 1), pl.ds(n_full, tail)]
                        fixq = jnp.where(
                            lane_t == lb - n_full, _phi(srcq) * s, winq)
                        bout[k, pl.ds(r, 1), pl.ds(n_full, tail)] = fixq
            out_cp(c, k).start()

            @pl.when(c + _NBUF < n_chunks)
            def _():
                in_cp(c + _NBUF, k).start()

        return carry

    lax.fori_loop(0, n_chunks // _NBUF, outer, 0)
    for k in range(_NBUF):
        out_cp(n_chunks - _NBUF + k, k).wait()


def kernel(cosine, labels, s):
    n_rows, n_cols = cosine.shape
    lab = labels.astype(jnp.int32)
    s_arr = jnp.asarray(s, jnp.float32).reshape(1, 1)

    return pl.pallas_call(
        _stream_body,
        in_specs=[
            pl.BlockSpec(memory_space=pltpu.SMEM),
            pl.BlockSpec(memory_space=pltpu.SMEM),
            pl.BlockSpec(memory_space=pltpu.HBM),
        ],
        out_specs=pl.BlockSpec(memory_space=pltpu.HBM),
        out_shape=jax.ShapeDtypeStruct((n_rows, n_cols), cosine.dtype),
        scratch_shapes=[
            pltpu.VMEM((_NBUF, _CH, n_cols), jnp.float32),
            pltpu.VMEM((_NBUF, _CH, n_cols), jnp.float32),
            pltpu.SemaphoreType.DMA((_NBUF,)),
            pltpu.SemaphoreType.DMA((_NBUF,)),
        ],
    )(s_arr, lab, cosine)
